# Initial kernel scaffold; baseline (speedup 1.0000x reference)
import jax
import jax.numpy as jnp
from jax.experimental import pallas as pl


def kernel(query, value, rand_matrix, seed):
    raise NotImplementedError("write your pallas kernel here")


# speedup vs baseline: 25.6154x; 25.6154x over previous
import math

import jax
import jax.numpy as jnp
from jax.experimental import pallas as pl

B, H, L, D_K, ROUNDS, BL = 1, 16, 2048, 64, 2, 64
BH = B * H
CH = 2 * BL
W = 2 * CH
NB2 = L // CH
NBUCK = 2 * NB2
NEG_BIG = -1000000000.0
NEG_SELF = -100000.0
LN2 = math.log(2.0)


def _iota(shape, dim):
    return jax.lax.broadcasted_iota(jnp.int32, shape, dim).astype(jnp.float32)


def _dotT(a, b, precision=jax.lax.Precision.HIGHEST):
    return jax.lax.dot_general(a, b, (((0,), (0,)), ((), ())),
                               precision=precision,
                               preferred_element_type=jnp.float32)


def _dot(a, b, precision=jax.lax.Precision.HIGHEST):
    return jax.lax.dot_general(a, b, (((1,), (0,)), ((), ())),
                               precision=precision,
                               preferred_element_type=jnp.float32)


def _argmax_lanes(x):
    vmax = jnp.max(x, axis=1, keepdims=True)
    cols = x.shape[1]
    idx = jnp.where(x == vmax, _iota(x.shape, 1), float(cols))
    return jnp.min(idx, axis=1, keepdims=True)


def _counting_sort(hash_col, tril_ch, tril_nb, triu_bk):
    oh = (hash_col == _iota((L, NBUCK), 1)).astype(jnp.float32)
    ranks, totals = [], []
    for n in range(NB2):
        blk = oh[n * CH:(n + 1) * CH]
        ranks.append(_dot(tril_ch, blk))
        totals.append(jnp.sum(blk, axis=0, keepdims=True))
    bt = jnp.concatenate(totals, axis=0)
    off = _dot(tril_nb, bt)
    counts = jnp.sum(bt, axis=0, keepdims=True)
    base = _dot(counts, triu_bk)
    pos = []
    for n in range(NB2):
        blk = oh[n * CH:(n + 1) * CH]
        p = jnp.sum(blk * (base + off[n:n + 1, :] + ranks[n]),
                    axis=1, keepdims=True)
        pos.append(p)
    return jnp.concatenate(pos, axis=0)


def _lsh_body(q_ref, v_ref, rm_ref, o_ref):
    q = q_ref[...]
    v = v_ref[...]
    rm = rm_ref[...]

    qn = q / jnp.sqrt(jnp.sum(q * q, axis=1, keepdims=True))
    rmn = rm / jnp.sqrt(jnp.sum(rm * rm, axis=0, keepdims=True))
    h = _dot(qn, rmn, precision=jax.lax.Precision.DEFAULT)

    tril_ch = (_iota((CH, CH), 0) > _iota((CH, CH), 1)).astype(jnp.float32)
    tril_nb = (_iota((NB2, NB2), 0) > _iota((NB2, NB2), 1)).astype(jnp.float32)
    triu_bk = (_iota((NBUCK, NBUCK), 0) < _iota((NBUCK, NBUCK), 1)).astype(jnp.float32)
    eye_ch = (_iota((CH, CH), 0) == _iota((CH, CH), 1)).astype(jnp.float32)

    hashes, poss, chunks = [], [], []
    for r in range(ROUNDS):
        hr = h[:, r * NB2:(r + 1) * NB2]
        hcat = jnp.concatenate([hr, -hr], axis=1)
        hsh = _argmax_lanes(hcat)
        pos = _counting_sort(hsh, tril_ch, tril_nb, triu_bk)
        hashes.append(hsh)
        poss.append(pos)
        chunks.append(jnp.floor(pos * (1.0 / CH)))

    payload = jnp.concatenate([qn, v], axis=1)
    meta = jnp.concatenate(
        [_iota((L, 1), 0), hashes[0], hashes[1], chunks[0], chunks[1],
         jnp.zeros((L, 3), jnp.float32)], axis=1)

    round_stats = []
    for r in range(ROUNDS):
        pos = poss[r]
        SQ, SV, SM, SMT, ES = [], [], [], [], []
        for n in range(NB2):
            E = (pos == (n * CH + _iota((L, CH), 1))).astype(jnp.float32)
            sp = _dotT(E, payload)
            sm = _dotT(E, meta)
            ES.append(E)
            SQ.append(sp[:, :D_K])
            SV.append(sp[:, D_K:])
            SM.append(sm)
            SMT.append(_dotT(sm, eye_ch))
        out_r = jnp.zeros((L, D_K + 8), jnp.float32)
        for n in range(NB2):
            p = (n - 1) % NB2
            Q = SQ[n]
            K = jnp.concatenate([SQ[p], SQ[n]], axis=0)
            V = jnp.concatenate([SV[p], SV[n]], axis=0)
            S = jax.lax.dot_general(Q, K, (((1,), (1,)), ((), ())),
                                    precision=jax.lax.Precision.HIGHEST,
                                    preferred_element_type=jnp.float32)
            S = S * (1.0 / math.sqrt(D_K))
            qi = SM[n][:, 0:1]
            sh = SM[n][:, 1 + r:2 + r]
            qo = SM[n][:, 4 - r:5 - r]
            wT = jnp.concatenate([SMT[p], SMT[n]], axis=1)
            ki = wT[0:1, :]
            kh = wT[1 + r:2 + r, :]
            ko = wT[4 - r:5 - r, :]
            S = jnp.where(sh != kh, NEG_BIG, S)
            S = jnp.where(qi < ki, NEG_BIG, S)
            S = jnp.where(qi == ki, NEG_SELF, S)
            qom1 = qo - 1.0
            qom1 = jnp.where(qom1 < 0.0, qom1 + float(NB2), qom1)
            dup = (ko == qo) | (ko == qom1)
            S = jnp.where(dup, S - LN2, S)
            m = jnp.max(S, axis=1, keepdims=True)
            wgt = jnp.exp(S - m)
            lsum = jnp.sum(wgt, axis=1, keepdims=True)
            acc = _dot(wgt, V)
            stat = jnp.concatenate(
                [acc, m, lsum, jnp.zeros((CH, 6), jnp.float32)], axis=1)
            out_r = out_r + _dot(ES[n], stat)
        round_stats.append(out_r)

    s0, s1 = round_stats
    a0, m0, l0 = s0[:, :D_K], s0[:, D_K:D_K + 1], s0[:, D_K + 1:D_K + 2]
    a1, m1, l1 = s1[:, :D_K], s1[:, D_K:D_K + 1], s1[:, D_K + 1:D_K + 2]
    mm = jnp.maximum(m0, m1)
    e0 = jnp.exp(m0 - mm)
    e1 = jnp.exp(m1 - mm)
    z = l0 * e0 + l1 * e1
    o_ref[...] = (a0 * e0 + a1 * e1) / z


def _run(q2, v2, rm2, interpret=False):
    return pl.pallas_call(
        _lsh_body,
        grid=(BH,),
        in_specs=[
            pl.BlockSpec((L, D_K), lambda i: (i, 0)),
            pl.BlockSpec((L, D_K), lambda i: (i, 0)),
            pl.BlockSpec((D_K, NBUCK), lambda i: (i, 0)),
        ],
        out_specs=pl.BlockSpec((L, D_K), lambda i: (i, 0)),
        out_shape=jax.ShapeDtypeStruct((BH * L, D_K), jnp.float32),
        interpret=interpret,
    )(q2, v2, rm2)


def kernel(query, value, rand_matrix, seed):
    q2 = query.reshape(BH * L, D_K)
    v2 = value.reshape(BH * L, D_K)
    rm2 = rand_matrix.reshape(BH * D_K, ROUNDS * NB2)
    out = _run(q2, v2, rm2)
    return out.reshape(B, H, L, D_K)


# speedup vs baseline: 52.9764x; 2.0681x over previous
import math

import jax
import jax.numpy as jnp
from jax.experimental import pallas as pl
from jax.experimental.pallas import tpu as pltpu

B, H, L, D_K, ROUNDS, BL = 1, 16, 2048, 64, 2, 64
BH = B * H
CH = 2 * BL
W = 2 * CH
NB2 = L // CH
NBUCK = 2 * NB2
NEG_BIG = -1000000000.0
NEG_SELF = -100000.0
LN2 = math.log(2.0)


def _iota(shape, dim):
    return jax.lax.broadcasted_iota(jnp.int32, shape, dim).astype(jnp.float32)


def _split2(x):
    hi = x.astype(jnp.bfloat16).astype(jnp.float32)
    return hi, x - hi


def _dotT(a, b, precision=jax.lax.Precision.DEFAULT):
    return jax.lax.dot_general(a, b, (((0,), (0,)), ((), ())),
                               precision=precision,
                               preferred_element_type=jnp.float32)


def _dot(a, b, precision=jax.lax.Precision.DEFAULT):
    return jax.lax.dot_general(a, b, (((1,), (0,)), ((), ())),
                               precision=precision,
                               preferred_element_type=jnp.float32)


def _argmax_lanes(x):
    vmax = jnp.max(x, axis=1, keepdims=True)
    cols = x.shape[1]
    idx = jnp.where(x == vmax, _iota(x.shape, 1), float(cols))
    return jnp.min(idx, axis=1, keepdims=True)


def _counting_sort(hash_col, tril_ch, tril_nb, triu_bk):
    oh = (hash_col == _iota((L, NBUCK), 1)).astype(jnp.float32)
    ranks, totals = [], []
    for n in range(NB2):
        blk = oh[n * CH:(n + 1) * CH]
        ranks.append(_dot(tril_ch, blk))
        totals.append(jnp.sum(blk, axis=0, keepdims=True))
    bt = jnp.concatenate(totals, axis=0)
    off = _dot(tril_nb, bt)
    counts = jnp.sum(bt, axis=0, keepdims=True)
    base = _dot(counts, triu_bk, precision=jax.lax.Precision.HIGHEST)
    pos = []
    for n in range(NB2):
        blk = oh[n * CH:(n + 1) * CH]
        p = jnp.sum(blk * (base + off[n:n + 1, :] + ranks[n]),
                    axis=1, keepdims=True)
        pos.append(p)
    return jnp.concatenate(pos, axis=0)


def _lsh_body(q_ref, v_ref, rm_ref, o_ref):
    q = q_ref[...]
    v = v_ref[...]
    rm = rm_ref[...]

    qn = q / jnp.sqrt(jnp.sum(q * q, axis=1, keepdims=True))
    rmn = rm / jnp.sqrt(jnp.sum(rm * rm, axis=0, keepdims=True))
    h = _dot(qn, rmn)

    tril_ch = (_iota((CH, CH), 0) > _iota((CH, CH), 1)).astype(jnp.float32)
    tril_nb = (_iota((NB2, NB2), 0) > _iota((NB2, NB2), 1)).astype(jnp.float32)
    triu_bk = (_iota((NBUCK, NBUCK), 0) < _iota((NBUCK, NBUCK), 1)).astype(jnp.float32)
    eye_ch = (_iota((CH, CH), 0) == _iota((CH, CH), 1)).astype(jnp.float32)

    hashes, poss, chunks = [], [], []
    for r in range(ROUNDS):
        hr = h[:, r * NB2:(r + 1) * NB2]
        hcat = jnp.concatenate([hr, -hr], axis=1)
        hsh = _argmax_lanes(hcat)
        pos = _counting_sort(hsh, tril_ch, tril_nb, triu_bk)
        hashes.append(hsh)
        poss.append(pos)
        chunks.append(jnp.floor(pos * (1.0 / CH)))

    ph, plo = _split2(jnp.concatenate([qn, v], axis=1))
    payload = jnp.concatenate([ph, plo], axis=1)
    meta1 = jnp.concatenate(
        [_iota((L, 1), 0), hashes[0], hashes[1], chunks[0], chunks[1],
         jnp.zeros((L, 3), jnp.float32)], axis=1)
    mh, mlo = _split2(meta1)
    meta = jnp.concatenate([mh, mlo], axis=1)

    round_stats = []
    for r in range(ROUNDS):
        pos = poss[r]
        SQ, SV, SM, SMT, ES = [], [], [], [], []
        for n in range(NB2):
            E = (pos == (n * CH + _iota((L, CH), 1))).astype(jnp.float32)
            sp2 = _dotT(E, payload)
            sp = sp2[:, :2 * D_K] + sp2[:, 2 * D_K:]
            sm2 = _dotT(E, meta)
            sm = sm2[:, :8] + sm2[:, 8:]
            ES.append(E)
            SQ.append(sp[:, :D_K])
            SV.append(sp[:, D_K:])
            SM.append(sm)
            SMT.append(_dotT(sm, eye_ch,
                             precision=jax.lax.Precision.HIGHEST))
        out_acc = jnp.zeros((L, 2 * (D_K + 8)), jnp.float32)
        for n in range(NB2):
            p = (n - 1) % NB2
            Q = SQ[n]
            K = jnp.concatenate([SQ[p], SQ[n]], axis=0)
            V = jnp.concatenate([SV[p], SV[n]], axis=0)
            S = jax.lax.dot_general(Q, K, (((1,), (1,)), ((), ())),
                                    preferred_element_type=jnp.float32)
            S = S * (1.0 / math.sqrt(D_K))
            qi = SM[n][:, 0:1]
            sh = SM[n][:, 1 + r:2 + r]
            qo = SM[n][:, 4 - r:5 - r]
            wT = jnp.concatenate([SMT[p], SMT[n]], axis=1)
            ki = wT[0:1, :]
            kh = wT[1 + r:2 + r, :]
            ko = wT[4 - r:5 - r, :]
            S = jnp.where(sh != kh, NEG_BIG, S)
            S = jnp.where(qi < ki, NEG_BIG, S)
            S = jnp.where(qi == ki, NEG_SELF, S)
            qom1 = qo - 1.0
            qom1 = jnp.where(qom1 < 0.0, qom1 + float(NB2), qom1)
            dup = (ko == qo) | (ko == qom1)
            S = jnp.where(dup, S - LN2, S)
            m = jnp.max(S, axis=1, keepdims=True)
            wgt = jnp.exp(S - m)
            lsum = jnp.sum(wgt, axis=1, keepdims=True)
            acc = _dot(wgt, V)
            sth, stl = _split2(jnp.concatenate(
                [acc, m, lsum, jnp.zeros((CH, 6), jnp.float32)], axis=1))
            stat = jnp.concatenate([sth, stl], axis=1)
            out_acc = out_acc + _dot(ES[n], stat)
        out_r = out_acc[:, :D_K + 8] + out_acc[:, D_K + 8:]
        round_stats.append(out_r)

    s0, s1 = round_stats
    a0, m0, l0 = s0[:, :D_K], s0[:, D_K:D_K + 1], s0[:, D_K + 1:D_K + 2]
    a1, m1, l1 = s1[:, :D_K], s1[:, D_K:D_K + 1], s1[:, D_K + 1:D_K + 2]
    mm = jnp.maximum(m0, m1)
    e0 = jnp.exp(m0 - mm)
    e1 = jnp.exp(m1 - mm)
    z = l0 * e0 + l1 * e1
    o_ref[...] = (a0 * e0 + a1 * e1) / z


def _run(q2, v2, rm2, interpret=False):
    return pl.pallas_call(
        _lsh_body,
        grid=(BH,),
        in_specs=[
            pl.BlockSpec((L, D_K), lambda i: (i, 0)),
            pl.BlockSpec((L, D_K), lambda i: (i, 0)),
            pl.BlockSpec((D_K, NBUCK), lambda i: (i, 0)),
        ],
        out_specs=pl.BlockSpec((L, D_K), lambda i: (i, 0)),
        out_shape=jax.ShapeDtypeStruct((BH * L, D_K), jnp.float32),
        compiler_params=pltpu.CompilerParams(
            dimension_semantics=("parallel",)),
        interpret=interpret,
    )(q2, v2, rm2)


def kernel(query, value, rand_matrix, seed):
    q2 = query.reshape(BH * L, D_K)
    v2 = value.reshape(BH * L, D_K)
    rm2 = rand_matrix.reshape(BH * D_K, ROUNDS * NB2)
    out = _run(q2, v2, rm2)
    return out.reshape(B, H, L, D_K)
